# Initial kernel scaffold; baseline (speedup 1.0000x reference)
import jax
import jax.numpy as jnp
from jax.experimental import pallas as pl


def kernel(x, adj_t, node_i, node_j, W1, b1, W2, b2, W3, b3):
    raise NotImplementedError("write your pallas kernel here")


# speedup vs baseline: 10.8508x; 10.8508x over previous
import functools

import jax
import jax.numpy as jnp
from jax import lax
from jax.experimental import pallas as pl
from jax.experimental.pallas import tpu as pltpu
from jax.experimental.pallas import tpu_sc as plsc

N = 10000
E = 320000
D = 128
P = 16384

NC = 2
NS = 16
NW = NC * NS

EPW = E // NW
EK = 80
NCH = EPW // EK
RPT = 640
RPT_LAST = N - RPT * (NS - 1)
PPW = P // NW
PK = 128
PCH = PPW // PK

_mesh = plsc.VectorSubcoreMesh(
    core_axis_name="c", subcore_axis_name="s", num_cores=NC, num_subcores=NS
)


@functools.partial(
    pl.kernel,
    out_type=jax.ShapeDtypeStruct((NC, N, D), jnp.float32),
    mesh=_mesh,
    scratch_types=[
        pltpu.VMEM((EK,), jnp.int32),
        pltpu.VMEM((EK,), jnp.float32),
        pltpu.VMEM((RPT,), jnp.float32),
        pltpu.VMEM((RPT, D), jnp.float32),
        pltpu.VMEM_SHARED((N,), jnp.float32),
    ],
)
def _sc_degree(dst_h, out_h, didx, onesv, vbuf, brd, acc):
    c = lax.axis_index("c")
    s = lax.axis_index("s")
    ebase = (c * NS + s) * EPW
    r0 = pl.multiple_of(s * RPT, 8)

    def zero(g, carry):
        vbuf[pl.ds(g * 16, 16)] = jnp.zeros((16,), jnp.float32)
        return carry

    lax.fori_loop(0, RPT // 16, zero, 0)

    @pl.when(s < NS - 1)
    def _():
        pltpu.sync_copy(vbuf, acc.at[pl.ds(r0, RPT)])

    @pl.when(s == NS - 1)
    def _():
        pltpu.sync_copy(vbuf.at[pl.ds(0, RPT_LAST)], acc.at[pl.ds(r0, RPT_LAST)])

    for i in range(EK // 16):
        onesv[pl.ds(i * 16, 16)] = jnp.ones((16,), jnp.float32)
    plsc.subcore_barrier()

    def body(k, carry):
        off = pl.multiple_of(ebase + k * EK, 8)
        pltpu.sync_copy(dst_h.at[pl.ds(off, EK)], didx)
        pltpu.sync_copy(onesv, acc.at[didx], add=True)
        return carry

    lax.fori_loop(0, NCH, body, 0)
    plsc.subcore_barrier()

    def bcast(g, carry):
        vec = vbuf[pl.ds(g * 16, 16)]
        for l in range(16):
            row = jnp.broadcast_to(vec[l], (16,))
            for j in range(D // 16):
                brd[g * 16 + l, pl.ds(j * 16, 16)] = row
        return carry

    @pl.when(s < NS - 1)
    def _():
        pltpu.sync_copy(acc.at[pl.ds(r0, RPT)], vbuf)
        lax.fori_loop(0, RPT // 16, bcast, 0)
        pltpu.sync_copy(brd, out_h.at[c, pl.ds(r0, RPT)])

    @pl.when(s == NS - 1)
    def _():
        pltpu.sync_copy(acc.at[pl.ds(r0, RPT_LAST)], vbuf.at[pl.ds(0, RPT_LAST)])
        lax.fori_loop(0, RPT_LAST // 16, bcast, 0)
        pltpu.sync_copy(brd.at[pl.ds(0, RPT_LAST)], out_h.at[c, pl.ds(r0, RPT_LAST)])


@functools.partial(
    pl.kernel,
    out_type=jax.ShapeDtypeStruct((NC, N, D), jnp.float32),
    mesh=_mesh,
    scratch_types=[
        pltpu.VMEM((EK,), jnp.int32),
        pltpu.VMEM((EK,), jnp.int32),
        pltpu.VMEM((EK, D), jnp.float32),
        pltpu.SemaphoreType.DMA,
        pltpu.VMEM_SHARED((N, D), jnp.float32),
    ],
)
def _sc_scatter(z_h, src_h, dst_h, out_h, sidx, didx, rows, sem, acc):
    c = lax.axis_index("c")
    s = lax.axis_index("s")
    ebase = (c * NS + s) * EPW
    r0 = pl.multiple_of(s * RPT, 8)

    @pl.when(s < NS - 1)
    def _():
        pltpu.sync_copy(z_h.at[pl.ds(r0, RPT)], acc.at[pl.ds(r0, RPT)])

    @pl.when(s == NS - 1)
    def _():
        pltpu.sync_copy(z_h.at[pl.ds(r0, RPT_LAST)], acc.at[pl.ds(r0, RPT_LAST)])

    plsc.subcore_barrier()

    def body(k, carry):
        off = pl.multiple_of(ebase + k * EK, 8)
        pltpu.sync_copy(src_h.at[pl.ds(off, EK)], sidx)
        pltpu.sync_copy(dst_h.at[pl.ds(off, EK)], didx)
        pltpu.async_copy(z_h.at[sidx], rows, sem).wait()
        pltpu.sync_copy(rows, acc.at[didx], add=True)
        return carry

    lax.fori_loop(0, NCH, body, 0)
    plsc.subcore_barrier()

    @pl.when(s < NS - 1)
    def _():
        pltpu.sync_copy(acc.at[pl.ds(r0, RPT)], out_h.at[c, pl.ds(r0, RPT)])

    @pl.when(s == NS - 1)
    def _():
        pltpu.sync_copy(acc.at[pl.ds(r0, RPT_LAST)], out_h.at[c, pl.ds(r0, RPT_LAST)])


@functools.partial(
    pl.kernel,
    out_type=(
        jax.ShapeDtypeStruct((P, D), jnp.float32),
        jax.ShapeDtypeStruct((P, D), jnp.float32),
    ),
    mesh=_mesh,
    scratch_types=[
        pltpu.VMEM((PK,), jnp.int32),
        pltpu.VMEM((PK, D), jnp.float32),
        pltpu.SemaphoreType.DMA,
    ],
)
def _sc_pair_gather(h_h, ni_h, nj_h, hi_h, hj_h, idxv, rows, sem):
    c = lax.axis_index("c")
    s = lax.axis_index("s")
    pbase = (c * NS + s) * PPW

    def body(k, carry):
        off = pl.multiple_of(pbase + k * PK, 8)
        pltpu.sync_copy(ni_h.at[pl.ds(off, PK)], idxv)
        pltpu.async_copy(h_h.at[idxv], rows, sem).wait()
        pltpu.sync_copy(rows, hi_h.at[pl.ds(off, PK)])
        pltpu.sync_copy(nj_h.at[pl.ds(off, PK)], idxv)
        pltpu.async_copy(h_h.at[idxv], rows, sem).wait()
        pltpu.sync_copy(rows, hj_h.at[pl.ds(off, PK)])
        return carry

    lax.fori_loop(0, PCH, body, 0)


RB = 1000
GN = N // RB


def _dinv_block(deg_ref):
    return lax.rsqrt(1.0 + deg_ref[0, :, 0:1] + deg_ref[1, :, 0:1])


def _tc_first_body(x_ref, w_ref, deg_ref, z_ref):
    dvec = _dinv_block(deg_ref)
    z_ref[...] = (
        jnp.dot(x_ref[...], w_ref[...], preferred_element_type=jnp.float32) * dvec
    )


_tc_first = pl.pallas_call(
    _tc_first_body,
    grid=(GN,),
    in_specs=[
        pl.BlockSpec((RB, D), lambda i: (i, 0)),
        pl.BlockSpec((D, D), lambda i: (0, 0)),
        pl.BlockSpec((NC, RB, D), lambda i: (0, i, 0)),
    ],
    out_specs=pl.BlockSpec((RB, D), lambda i: (i, 0)),
    out_shape=jax.ShapeDtypeStruct((N, D), jnp.float32),
)


def _tc_mid_body(p_ref, z_ref, deg_ref, b_ref, w_ref, out_ref):
    dvec = _dinv_block(deg_ref)
    agg = p_ref[0] + p_ref[1] - z_ref[...]
    h = jnp.maximum(dvec * agg + b_ref[...], 0.0)
    out_ref[...] = (
        jnp.dot(h, w_ref[...], preferred_element_type=jnp.float32) * dvec
    )


_tc_mid = pl.pallas_call(
    _tc_mid_body,
    grid=(GN,),
    in_specs=[
        pl.BlockSpec((NC, RB, D), lambda i: (0, i, 0)),
        pl.BlockSpec((RB, D), lambda i: (i, 0)),
        pl.BlockSpec((NC, RB, D), lambda i: (0, i, 0)),
        pl.BlockSpec((1, D), lambda i: (0, 0)),
        pl.BlockSpec((D, D), lambda i: (0, 0)),
    ],
    out_specs=pl.BlockSpec((RB, D), lambda i: (i, 0)),
    out_shape=jax.ShapeDtypeStruct((N, D), jnp.float32),
)


def _tc_final_body(p_ref, z_ref, deg_ref, b_ref, out_ref):
    dvec = _dinv_block(deg_ref)
    agg = p_ref[0] + p_ref[1] - z_ref[...]
    out_ref[...] = dvec * agg + b_ref[...]


_tc_final = pl.pallas_call(
    _tc_final_body,
    grid=(GN,),
    in_specs=[
        pl.BlockSpec((NC, RB, D), lambda i: (0, i, 0)),
        pl.BlockSpec((RB, D), lambda i: (i, 0)),
        pl.BlockSpec((NC, RB, D), lambda i: (0, i, 0)),
        pl.BlockSpec((1, D), lambda i: (0, 0)),
    ],
    out_specs=pl.BlockSpec((RB, D), lambda i: (i, 0)),
    out_shape=jax.ShapeDtypeStruct((N, D), jnp.float32),
)


PB = 2048


def _tc_dot_body(hi_ref, hj_ref, o_ref):
    o_ref[...] = jnp.sum(hi_ref[...] * hj_ref[...], axis=1)


_tc_dot = pl.pallas_call(
    _tc_dot_body,
    grid=(P // PB,),
    in_specs=[
        pl.BlockSpec((PB, D), lambda i: (i, 0)),
        pl.BlockSpec((PB, D), lambda i: (i, 0)),
    ],
    out_specs=pl.BlockSpec((PB,), lambda i: (i,)),
    out_shape=jax.ShapeDtypeStruct((P,), jnp.float32),
)


def kernel(x, adj_t, node_i, node_j, W1, b1, W2, b2, W3, b3):
    src = adj_t[0]
    dst = adj_t[1]
    degbuf = _sc_degree(dst)

    z1 = _tc_first(x, W1, degbuf)
    p1 = _sc_scatter(z1, src, dst)
    z2 = _tc_mid(p1, z1, degbuf, b1.reshape(1, D), W2)
    p2 = _sc_scatter(z2, src, dst)
    z3 = _tc_mid(p2, z2, degbuf, b2.reshape(1, D), W3)
    p3 = _sc_scatter(z3, src, dst)
    h3 = _tc_final(p3, z3, degbuf, b3.reshape(1, D))

    hi, hj = _sc_pair_gather(h3, node_i, node_j)
    return _tc_dot(hi, hj)


# speedup vs baseline: 26.0369x; 2.3995x over previous
import functools

import jax
import jax.numpy as jnp
from jax import lax
from jax.experimental import pallas as pl
from jax.experimental.pallas import tpu as pltpu
from jax.experimental.pallas import tpu_sc as plsc

N = 10000
E = 320000
D = 128
P = 16384

NC = 2
NS = 16
NW = NC * NS

EPW = E // NW
EK = 40
NCH = EPW // EK
NB = 5
NGRP = NCH // NB
EKD = 80
NCHD = EPW // EKD
NBD = 5
NGRPD = NCHD // NBD
RPT = 640
RPT_LAST = N - RPT * (NS - 1)
PPW = P // NW
PK = 128
PCH = PPW // PK

_mesh = plsc.VectorSubcoreMesh(
    core_axis_name="c", subcore_axis_name="s", num_cores=NC, num_subcores=NS
)


@functools.partial(
    pl.kernel,
    out_type=jax.ShapeDtypeStruct((NC, N, D), jnp.float32),
    mesh=_mesh,
    scratch_types=[
        pltpu.VMEM((EPW,), jnp.int32),
        pltpu.VMEM((EKD,), jnp.float32),
        pltpu.VMEM((RPT,), jnp.float32),
        pltpu.VMEM((RPT, D), jnp.float32),
        pltpu.SemaphoreType.DMA,
        pltpu.VMEM_SHARED((N,), jnp.float32),
    ],
)
def _sc_degree(dst_h, out_h, didx, onesv, vbuf, brd, sem, acc):
    c = lax.axis_index("c")
    s = lax.axis_index("s")
    tid = c * NS + s
    r0 = pl.multiple_of(s * RPT, 8)

    def zero(g, carry):
        vbuf[pl.ds(g * 16, 16)] = jnp.zeros((16,), jnp.float32)
        return carry

    lax.fori_loop(0, RPT // 16, zero, 0)

    @pl.when(s < NS - 1)
    def _():
        pltpu.sync_copy(vbuf, acc.at[pl.ds(r0, RPT)])

    @pl.when(s == NS - 1)
    def _():
        pltpu.sync_copy(vbuf.at[pl.ds(0, RPT_LAST)], acc.at[pl.ds(r0, RPT_LAST)])

    for i in range(EKD // 16):
        onesv[pl.ds(i * 16, 16)] = jnp.ones((16,), jnp.float32)
    pltpu.sync_copy(dst_h.at[tid], didx)
    plsc.subcore_barrier()

    def body(g, carry):
        for b in range(NBD):
            k = g * NBD + b
            pltpu.async_copy(onesv, acc.at[didx.at[pl.ds(k * EKD, EKD)]], sem, add=True)
        for b in range(NBD):
            k = g * NBD + b
            pltpu.make_async_copy(onesv, acc.at[didx.at[pl.ds(k * EKD, EKD)]], sem).wait()
        return carry

    lax.fori_loop(0, NGRPD, body, 0)
    plsc.subcore_barrier()

    def bcast(g, carry):
        vec = vbuf[pl.ds(g * 16, 16)]
        for l in range(16):
            row = jnp.broadcast_to(vec[l], (16,))
            for j in range(D // 16):
                brd[g * 16 + l, pl.ds(j * 16, 16)] = row
        return carry

    @pl.when(s < NS - 1)
    def _():
        pltpu.sync_copy(acc.at[pl.ds(r0, RPT)], vbuf)
        lax.fori_loop(0, RPT // 16, bcast, 0)
        pltpu.sync_copy(brd, out_h.at[c, pl.ds(r0, RPT)])

    @pl.when(s == NS - 1)
    def _():
        pltpu.sync_copy(acc.at[pl.ds(r0, RPT_LAST)], vbuf.at[pl.ds(0, RPT_LAST)])
        lax.fori_loop(0, RPT_LAST // 16, bcast, 0)
        pltpu.sync_copy(brd.at[pl.ds(0, RPT_LAST)], out_h.at[c, pl.ds(r0, RPT_LAST)])


@functools.partial(
    pl.kernel,
    out_type=jax.ShapeDtypeStruct((NC, N, D), jnp.float32),
    mesh=_mesh,
    scratch_types=[
        pltpu.VMEM((EPW,), jnp.int32),
        pltpu.VMEM((EPW,), jnp.int32),
        pltpu.VMEM((NB, EK, D), jnp.float32),
        pltpu.SemaphoreType.DMA,
        pltpu.SemaphoreType.DMA,
        pltpu.VMEM_SHARED((N, D), jnp.float32),
    ],
)
def _sc_scatter(z_h, src_h, dst_h, out_h, sidx, didx, rows, sem_g, sem_s, acc):
    c = lax.axis_index("c")
    s = lax.axis_index("s")
    tid = c * NS + s
    r0 = pl.multiple_of(s * RPT, 8)

    pltpu.sync_copy(src_h.at[tid], sidx)
    pltpu.sync_copy(dst_h.at[tid], didx)

    @pl.when(s < NS - 1)
    def _():
        pltpu.sync_copy(z_h.at[pl.ds(r0, RPT)], acc.at[pl.ds(r0, RPT)])

    @pl.when(s == NS - 1)
    def _():
        pltpu.sync_copy(z_h.at[pl.ds(r0, RPT_LAST)], acc.at[pl.ds(r0, RPT_LAST)])

    plsc.subcore_barrier()

    for b in range(NB):
        pltpu.async_copy(z_h.at[sidx.at[pl.ds(b * EK, EK)]], rows.at[b], sem_g)

    def body(g, carry):
        for b in range(NB):
            k = g * NB + b
            pltpu.make_async_copy(
                z_h.at[sidx.at[pl.ds(k * EK, EK)]], rows.at[b], sem_g
            ).wait()
            pltpu.async_copy(rows.at[b], acc.at[didx.at[pl.ds(k * EK, EK)]], sem_s, add=True)
        for b in range(NB):
            k = g * NB + b
            pltpu.make_async_copy(
                rows.at[b], acc.at[didx.at[pl.ds(k * EK, EK)]], sem_s
            ).wait()
            kk = k + NB

            @pl.when(kk < NCH)
            def _():
                pltpu.async_copy(z_h.at[sidx.at[pl.ds(kk * EK, EK)]], rows.at[b], sem_g)

        return carry

    lax.fori_loop(0, NGRP, body, 0)
    plsc.subcore_barrier()

    @pl.when(s < NS - 1)
    def _():
        pltpu.sync_copy(acc.at[pl.ds(r0, RPT)], out_h.at[c, pl.ds(r0, RPT)])

    @pl.when(s == NS - 1)
    def _():
        pltpu.sync_copy(acc.at[pl.ds(r0, RPT_LAST)], out_h.at[c, pl.ds(r0, RPT_LAST)])


@functools.partial(
    pl.kernel,
    out_type=(
        jax.ShapeDtypeStruct((P, D), jnp.float32),
        jax.ShapeDtypeStruct((P, D), jnp.float32),
    ),
    mesh=_mesh,
    scratch_types=[
        pltpu.VMEM((PCH, PK), jnp.int32),
        pltpu.VMEM((PCH, PK), jnp.int32),
        pltpu.VMEM((PCH, PK, D), jnp.float32),
        pltpu.SemaphoreType.DMA,
        pltpu.SemaphoreType.DMA,
    ],
)
def _sc_pair_gather(h_h, ni_h, nj_h, hi_h, hj_h, iidx, jidx, rows, sem_g, sem_w):
    c = lax.axis_index("c")
    s = lax.axis_index("s")
    tid = c * NS + s
    pbase = pl.multiple_of(tid * PPW, 8)

    pltpu.sync_copy(ni_h.at[tid], iidx)
    pltpu.sync_copy(nj_h.at[tid], jidx)
    for k in range(PCH):
        pltpu.async_copy(h_h.at[iidx.at[k]], rows.at[k], sem_g)
    for k in range(PCH):
        pltpu.make_async_copy(h_h.at[iidx.at[k]], rows.at[k], sem_g).wait()
        pltpu.async_copy(rows.at[k], hi_h.at[pl.ds(pbase + k * PK, PK)], sem_w)
    for k in range(PCH):
        pltpu.make_async_copy(rows.at[k], hi_h.at[pl.ds(pbase + k * PK, PK)], sem_w).wait()
        pltpu.async_copy(h_h.at[jidx.at[k]], rows.at[k], sem_g)
    for k in range(PCH):
        pltpu.make_async_copy(h_h.at[jidx.at[k]], rows.at[k], sem_g).wait()
        pltpu.async_copy(rows.at[k], hj_h.at[pl.ds(pbase + k * PK, PK)], sem_w)
    for k in range(PCH):
        pltpu.make_async_copy(rows.at[k], hj_h.at[pl.ds(pbase + k * PK, PK)], sem_w).wait()


RB = 1000
GN = N // RB


def _dinv_block(deg_ref):
    return lax.rsqrt(1.0 + deg_ref[0, :, 0:1] + deg_ref[1, :, 0:1])


def _tc_first_body(x_ref, w_ref, deg_ref, z_ref):
    dvec = _dinv_block(deg_ref)
    z_ref[...] = (
        jnp.dot(x_ref[...], w_ref[...], preferred_element_type=jnp.float32) * dvec
    )


_tc_first = pl.pallas_call(
    _tc_first_body,
    grid=(GN,),
    in_specs=[
        pl.BlockSpec((RB, D), lambda i: (i, 0)),
        pl.BlockSpec((D, D), lambda i: (0, 0)),
        pl.BlockSpec((NC, RB, D), lambda i: (0, i, 0)),
    ],
    out_specs=pl.BlockSpec((RB, D), lambda i: (i, 0)),
    out_shape=jax.ShapeDtypeStruct((N, D), jnp.float32),
)


def _tc_mid_body(p_ref, z_ref, deg_ref, b_ref, w_ref, out_ref):
    dvec = _dinv_block(deg_ref)
    agg = p_ref[0] + p_ref[1] - z_ref[...]
    h = jnp.maximum(dvec * agg + b_ref[...], 0.0)
    out_ref[...] = (
        jnp.dot(h, w_ref[...], preferred_element_type=jnp.float32) * dvec
    )


_tc_mid = pl.pallas_call(
    _tc_mid_body,
    grid=(GN,),
    in_specs=[
        pl.BlockSpec((NC, RB, D), lambda i: (0, i, 0)),
        pl.BlockSpec((RB, D), lambda i: (i, 0)),
        pl.BlockSpec((NC, RB, D), lambda i: (0, i, 0)),
        pl.BlockSpec((1, D), lambda i: (0, 0)),
        pl.BlockSpec((D, D), lambda i: (0, 0)),
    ],
    out_specs=pl.BlockSpec((RB, D), lambda i: (i, 0)),
    out_shape=jax.ShapeDtypeStruct((N, D), jnp.float32),
)


def _tc_final_body(p_ref, z_ref, deg_ref, b_ref, out_ref):
    dvec = _dinv_block(deg_ref)
    agg = p_ref[0] + p_ref[1] - z_ref[...]
    out_ref[...] = dvec * agg + b_ref[...]


_tc_final = pl.pallas_call(
    _tc_final_body,
    grid=(GN,),
    in_specs=[
        pl.BlockSpec((NC, RB, D), lambda i: (0, i, 0)),
        pl.BlockSpec((RB, D), lambda i: (i, 0)),
        pl.BlockSpec((NC, RB, D), lambda i: (0, i, 0)),
        pl.BlockSpec((1, D), lambda i: (0, 0)),
    ],
    out_specs=pl.BlockSpec((RB, D), lambda i: (i, 0)),
    out_shape=jax.ShapeDtypeStruct((N, D), jnp.float32),
)


PB = 2048


def _tc_dot_body(hi_ref, hj_ref, o_ref):
    o_ref[...] = jnp.sum(hi_ref[...] * hj_ref[...], axis=1)


_tc_dot = pl.pallas_call(
    _tc_dot_body,
    grid=(P // PB,),
    in_specs=[
        pl.BlockSpec((PB, D), lambda i: (i, 0)),
        pl.BlockSpec((PB, D), lambda i: (i, 0)),
    ],
    out_specs=pl.BlockSpec((PB,), lambda i: (i,)),
    out_shape=jax.ShapeDtypeStruct((P,), jnp.float32),
)


def kernel(x, adj_t, node_i, node_j, W1, b1, W2, b2, W3, b3):
    src = adj_t[0].reshape(NW, EPW)
    dst = adj_t[1].reshape(NW, EPW)
    node_i = node_i.reshape(NW, PCH, PK)
    node_j = node_j.reshape(NW, PCH, PK)
    degbuf = _sc_degree(dst)

    z1 = _tc_first(x, W1, degbuf)
    p1 = _sc_scatter(z1, src, dst)
    z2 = _tc_mid(p1, z1, degbuf, b1.reshape(1, D), W2)
    p2 = _sc_scatter(z2, src, dst)
    z3 = _tc_mid(p2, z2, degbuf, b2.reshape(1, D), W3)
    p3 = _sc_scatter(z3, src, dst)
    h3 = _tc_final(p3, z3, degbuf, b3.reshape(1, D))

    hi, hj = _sc_pair_gather(h3, node_i, node_j)
    return _tc_dot(hi, hj)
